# baseline (speedup 1.0000x reference)
import functools

import jax
import jax.numpy as jnp
from jax import lax
from jax.experimental import pallas as pl
from jax.experimental.pallas import tpu as pltpu
from jax.experimental.pallas import tpu_sc as plsc

B = 1024
CTX = 20
EMB = 64
HID = 128
LANES = 16
VOCAB = 100000

_NC = 2
_NS = 16
_NW = _NC * _NS
_B_PER_W = B // _NW
_IDX_PER_W = _B_PER_W * CTX
_GCHUNK = 128
_NCHUNK = _IDX_PER_W // _GCHUNK


def _pool_body(emb_hbm, idx_hbm, out_hbm, idx_v, rows_v, pooled_v, sem):
    wid = lax.axis_index("s") * _NC + lax.axis_index("c")
    ibase = wid * _IDX_PER_W
    obase = wid * _B_PER_W

    pltpu.sync_copy(idx_hbm.at[pl.ds(ibase, _IDX_PER_W)], idx_v)

    copies = []
    for k in range(_NCHUNK):
        copies.append(pltpu.async_copy(
            emb_hbm.at[idx_v.at[pl.ds(k * _GCHUNK, _GCHUNK)]],
            rows_v.at[pl.ds(k * _GCHUNK, _GCHUNK)],
            sem,
        ))
    for c in copies:
        c.wait()

    def body(b, carry):
        for ch in range(EMB // LANES):
            sl = pl.ds(ch * LANES, LANES)
            acc = rows_v[b * CTX, sl]
            for c in range(1, CTX):
                acc = acc + rows_v[b * CTX + c, sl]
            pooled_v[b, sl] = acc
        return carry

    lax.fori_loop(0, _B_PER_W, body, None)
    pltpu.sync_copy(pooled_v, out_hbm.at[pl.ds(obase, _B_PER_W)])


_sc_pool = functools.partial(
    pl.kernel,
    mesh=plsc.VectorSubcoreMesh(core_axis_name="c", subcore_axis_name="s"),
    out_type=jax.ShapeDtypeStruct((B, EMB), jnp.float32),
    scratch_types=[
        pltpu.VMEM((_IDX_PER_W,), jnp.int32),
        pltpu.VMEM((_IDX_PER_W, EMB), jnp.float32),
        pltpu.VMEM((_B_PER_W, EMB), jnp.float32),
        pltpu.SemaphoreType.DMA,
    ],
    compiler_params=pltpu.CompilerParams(use_tc_tiling_on_sc=False),
)(_pool_body)


_BT = 8
_NB = B // _BT


def _mlp_body(pooled_ref, W1_ref, b1_ref, W2_ref, b2_ref, out_ref):
    h = jnp.dot(pooled_ref[...], W1_ref[...],
                preferred_element_type=jnp.float32)
    h = jnp.maximum(h + b1_ref[...], 0.0)
    out_ref[...] = jnp.dot(h, W2_ref[...],
                           preferred_element_type=jnp.float32) + b2_ref[...]


def _tc_mlp(pooled, W1, b1, W2, b2):
    return pl.pallas_call(
        _mlp_body,
        grid=(_NB,),
        in_specs=[
            pl.BlockSpec((_BT, EMB), lambda i: (i, 0)),
            pl.BlockSpec((EMB, HID), lambda i: (0, 0)),
            pl.BlockSpec((1, HID), lambda i: (0, 0)),
            pl.BlockSpec((HID, VOCAB), lambda i: (0, 0)),
            pl.BlockSpec((1, VOCAB), lambda i: (0, 0)),
        ],
        out_specs=pl.BlockSpec((_BT, VOCAB), lambda i: (i, 0)),
        out_shape=jax.ShapeDtypeStruct((B, VOCAB), jnp.float32),
        compiler_params=pltpu.CompilerParams(
            dimension_semantics=("arbitrary",),
            vmem_limit_bytes=63 * 1024 * 1024,
        ),
    )(pooled, W1, b1, W2, b2)


def kernel(inputs, emb, W1, b1, W2, b2):
    idx = inputs.reshape(-1).astype(jnp.int32)
    pooled = _sc_pool(emb, idx)
    return _tc_mlp(pooled, W1, b1.reshape(1, HID), W2, b2.reshape(1, -1))


# speedup vs baseline: 3.3271x; 3.3271x over previous
import functools

import jax
import jax.numpy as jnp
from jax import lax
from jax.experimental import pallas as pl
from jax.experimental.pallas import tpu as pltpu
from jax.experimental.pallas import tpu_sc as plsc

B = 1024
CTX = 20
EMB = 64
HID = 128
LANES = 16
VOCAB = 100000

_NC = 2
_NS = 16
_NW = _NC * _NS
_B_PER_W = B // _NW
_IDX_PER_W = _B_PER_W * CTX
_GCHUNK = 128
_NCHUNK = _IDX_PER_W // _GCHUNK


def _pool_body(emb_hbm, idx_hbm, out_hbm, idx_v, rows_v, pooled_v, sem):
    wid = lax.axis_index("s") * _NC + lax.axis_index("c")
    ibase = wid * _IDX_PER_W
    obase = wid * _B_PER_W

    pltpu.sync_copy(idx_hbm.at[pl.ds(ibase, _IDX_PER_W)], idx_v)

    copies = []
    for k in range(_NCHUNK):
        copies.append(pltpu.async_copy(
            emb_hbm.at[idx_v.at[pl.ds(k * _GCHUNK, _GCHUNK)]],
            rows_v.at[pl.ds(k * _GCHUNK, _GCHUNK)],
            sem,
        ))
    for c in copies:
        c.wait()

    def body(b, carry):
        for ch in range(EMB // LANES):
            sl = pl.ds(ch * LANES, LANES)
            acc = rows_v[b * CTX, sl]
            for c in range(1, CTX):
                acc = acc + rows_v[b * CTX + c, sl]
            pooled_v[b, sl] = acc
        return carry

    lax.fori_loop(0, _B_PER_W, body, None)
    pltpu.sync_copy(pooled_v, out_hbm.at[pl.ds(obase, _B_PER_W)])


_sc_pool = functools.partial(
    pl.kernel,
    mesh=plsc.VectorSubcoreMesh(core_axis_name="c", subcore_axis_name="s"),
    out_type=jax.ShapeDtypeStruct((B, EMB), jnp.float32),
    scratch_types=[
        pltpu.VMEM((_IDX_PER_W,), jnp.int32),
        pltpu.VMEM((_IDX_PER_W, EMB), jnp.float32),
        pltpu.VMEM((_B_PER_W, EMB), jnp.float32),
        pltpu.SemaphoreType.DMA,
    ],
    compiler_params=pltpu.CompilerParams(use_tc_tiling_on_sc=False),
)(_pool_body)


_VT = 2000
_NV = VOCAB // _VT


def _mlp_body(pooledT_ref, W1_ref, b1_ref, W2T_ref, b2_ref, outT_ref, hT_ref):
    @pl.when(pl.program_id(0) == 0)
    def _():
        hT = jax.lax.dot_general(
            W1_ref[...], pooledT_ref[...],
            dimension_numbers=(((0,), (0,)), ((), ())),
            preferred_element_type=jnp.float32)
        hT_ref[...] = jnp.maximum(hT + b1_ref[...], 0.0)

    outT_ref[...] = jnp.dot(W2T_ref[...], hT_ref[...],
                            preferred_element_type=jnp.float32) + b2_ref[...]


def _tc_mlp(pooledT, W1, b1, W2T, b2):
    return pl.pallas_call(
        _mlp_body,
        grid=(_NV,),
        in_specs=[
            pl.BlockSpec((EMB, B), lambda i: (0, 0)),
            pl.BlockSpec((EMB, HID), lambda i: (0, 0)),
            pl.BlockSpec((HID, 1), lambda i: (0, 0)),
            pl.BlockSpec((_VT, HID), lambda i: (i, 0)),
            pl.BlockSpec((_VT, 1), lambda i: (i, 0)),
        ],
        out_specs=pl.BlockSpec((_VT, B), lambda i: (i, 0)),
        out_shape=jax.ShapeDtypeStruct((VOCAB, B), jnp.float32),
        scratch_shapes=[pltpu.VMEM((HID, B), jnp.float32)],
        compiler_params=pltpu.CompilerParams(
            dimension_semantics=("arbitrary",)),
    )(pooledT, W1, b1, W2T, b2)


def kernel(inputs, emb, W1, b1, W2, b2):
    idx = inputs.reshape(-1).astype(jnp.int32)
    pooled = _sc_pool(emb, idx)
    outT = _tc_mlp(pooled.T, W1, b1.reshape(HID, 1), W2.T,
                   b2.reshape(VOCAB, 1))
    return outT.T


# speedup vs baseline: 3.3316x; 1.0013x over previous
import functools

import jax
import jax.numpy as jnp
from jax import lax
from jax.experimental import pallas as pl
from jax.experimental.pallas import tpu as pltpu
from jax.experimental.pallas import tpu_sc as plsc

B = 1024
CTX = 20
EMB = 64
HID = 128
LANES = 16
VOCAB = 100000

_NC = 2
_NS = 16
_NW = _NC * _NS
_B_PER_W = B // _NW
_IDX_PER_W = _B_PER_W * CTX
_GCHUNK = 128
_NCHUNK = _IDX_PER_W // _GCHUNK


def _pool_body(emb_hbm, idx_hbm, out_hbm, idx_v, big_v, rows_v, pooled_v, sem):
    wid = lax.axis_index("s") * _NC + lax.axis_index("c")
    ibase = wid * _IDX_PER_W
    obase = wid * _B_PER_W

    pltpu.sync_copy(idx_hbm.at[pl.ds(ibase, _IDX_PER_W)], idx_v)

    for j in range(_IDX_PER_W // LANES):
        sl = pl.ds(j * LANES, LANES)
        big_v[sl] = idx_v[sl] >> 1

    copies = []
    for k in range(_NCHUNK):
        copies.append(pltpu.async_copy(
            emb_hbm.at[big_v.at[pl.ds(k * _GCHUNK, _GCHUNK)]],
            rows_v.at[pl.ds(k * _GCHUNK, _GCHUNK)],
            sem,
        ))
    for c in copies:
        c.wait()

    def body(b, carry):
        k0 = b * CTX
        offa = (idx_v[pl.ds(k0, LANES)] & 1) * EMB
        offb = (idx_v[pl.ds(k0 + CTX - LANES, LANES)] & 1) * EMB
        for ch in range(EMB // LANES):
            acc = rows_v[k0, pl.ds(offa[0] + ch * LANES, LANES)]
            for c in range(1, CTX):
                off = offa[c] if c < LANES else offb[c - (CTX - LANES)]
                acc = acc + rows_v[k0 + c, pl.ds(off + ch * LANES, LANES)]
            pooled_v[b, pl.ds(ch * LANES, LANES)] = acc
        return carry

    lax.fori_loop(0, _B_PER_W, body, None)
    pltpu.sync_copy(pooled_v, out_hbm.at[pl.ds(obase, _B_PER_W)])


_sc_pool = functools.partial(
    pl.kernel,
    mesh=plsc.VectorSubcoreMesh(core_axis_name="c", subcore_axis_name="s"),
    out_type=jax.ShapeDtypeStruct((B, EMB), jnp.float32),
    scratch_types=[
        pltpu.VMEM((_IDX_PER_W,), jnp.int32),
        pltpu.VMEM((_IDX_PER_W,), jnp.int32),
        pltpu.VMEM((_IDX_PER_W, 2 * EMB), jnp.float32),
        pltpu.VMEM((_B_PER_W, EMB), jnp.float32),
        pltpu.SemaphoreType.DMA,
    ],
)(_pool_body)


_VT = 2000
_NV = VOCAB // _VT


def _mlp_body(pooledT_ref, W1_ref, b1_ref, W2T_ref, b2_ref, outT_ref, hT_ref):
    @pl.when(pl.program_id(0) == 0)
    def _():
        hT = jax.lax.dot_general(
            W1_ref[...], pooledT_ref[...],
            dimension_numbers=(((0,), (0,)), ((), ())),
            preferred_element_type=jnp.float32)
        hT_ref[...] = jnp.maximum(hT + b1_ref[...], 0.0)

    outT_ref[...] = jnp.dot(W2T_ref[...], hT_ref[...],
                            preferred_element_type=jnp.float32) + b2_ref[...]


def _tc_mlp(pooledT, W1, b1, W2T, b2):
    return pl.pallas_call(
        _mlp_body,
        grid=(_NV,),
        in_specs=[
            pl.BlockSpec((EMB, B), lambda i: (0, 0)),
            pl.BlockSpec((EMB, HID), lambda i: (0, 0)),
            pl.BlockSpec((HID, 1), lambda i: (0, 0)),
            pl.BlockSpec((_VT, HID), lambda i: (i, 0)),
            pl.BlockSpec((_VT, 1), lambda i: (i, 0)),
        ],
        out_specs=pl.BlockSpec((_VT, B), lambda i: (i, 0)),
        out_shape=jax.ShapeDtypeStruct((VOCAB, B), jnp.float32),
        scratch_shapes=[pltpu.VMEM((HID, B), jnp.float32)],
        compiler_params=pltpu.CompilerParams(
            dimension_semantics=("arbitrary",)),
    )(pooledT, W1, b1, W2T, b2)


def kernel(inputs, emb, W1, b1, W2, b2):
    idx = inputs.reshape(-1).astype(jnp.int32)
    pooled = _sc_pool(emb.reshape(VOCAB // 2, 2 * EMB), idx)
    outT = _tc_mlp(pooled.T, W1, b1.reshape(HID, 1), W2.T,
                   b2.reshape(VOCAB, 1))
    return outT.T
